# Initial kernel scaffold; baseline (speedup 1.0000x reference)
import jax
import jax.numpy as jnp
from jax.experimental import pallas as pl


def kernel(x, edge_index, W0, b0, W1, b1, W2, b2, W3, b3, W4, b4, W5, b5, W6, b6, W7, b7):
    raise NotImplementedError("write your pallas kernel here")


# speedup vs baseline: 19.0322x; 19.0322x over previous
import functools

import jax
import jax.numpy as jnp
from jax import lax
from jax.experimental import pallas as pl
from jax.experimental.pallas import tpu as pltpu
from jax.experimental.pallas import tpu_sc as plsc

N = 10000
E = 320000
D = 16
NP = 10240
NTILES = 32
CH = 128
NCH = 80
EPT = NCH * CH
EPAD = NTILES * EPT
DUMMY = N + 16
RPT = NP // 16
PK = NP // 8


def _sc_propagate_body(g_hbm, srcs_hbm, dsts_hbm, zeros_hbm, out_hbm,
                       src_v, dst_v, msg_v, agg_sh):
    c = lax.axis_index("c")
    s = lax.axis_index("s")
    tid = s * 2 + c
    pltpu.sync_copy(srcs_hbm.at[tid], src_v)
    pltpu.sync_copy(dsts_hbm.at[tid], dst_v)
    pltpu.sync_copy(zeros_hbm.at[pl.ds(s * RPT, RPT)],
                    agg_sh.at[pl.ds(s * RPT, RPT)])
    plsc.subcore_barrier()

    def body(j, carry):
        pltpu.sync_copy(g_hbm.at[src_v.at[j]], msg_v)
        pltpu.sync_copy(msg_v, agg_sh.at[dst_v.at[j]], add=True)
        return carry

    lax.fori_loop(0, NCH, body, 0)
    plsc.subcore_barrier()
    pltpu.sync_copy(agg_sh.at[pl.ds(s * RPT, RPT)],
                    out_hbm.at[c].at[pl.ds(s * RPT, RPT)])


_sc_propagate = functools.partial(
    pl.kernel,
    out_type=jax.ShapeDtypeStruct((2, NP, D), jnp.float32),
    mesh=plsc.VectorSubcoreMesh(core_axis_name="c", subcore_axis_name="s"),
    scratch_types=[
        pltpu.VMEM((NCH, CH), jnp.int32),
        pltpu.VMEM((NCH, CH), jnp.int32),
        pltpu.VMEM((CH, D), jnp.float32),
        pltpu.VMEM_SHARED((NP, D), jnp.float32),
    ],
    compiler_params=pltpu.CompilerParams(use_tc_tiling_on_sc=False),
)(_sc_propagate_body)


def _propagate(g, srcs, dsts, zeros):
    return _sc_propagate(g, srcs, dsts, zeros)


def _tc_first_body(a0_ref, a1_ref, x_ref, w_ref, dinv_ref, g_ref):
    dinv = lax.rsqrt(a0_ref[...] + a1_ref[...] + 1.0)
    dinv_ref[...] = dinv
    g_ref[...] = jnp.dot(x_ref[...], w_ref[...],
                         preferred_element_type=jnp.float32) * dinv


def _tc_mid_body(s0_ref, s1_ref, g_ref, dinv_ref, b_ref, w_ref, out_ref):
    dinv = dinv_ref[...]
    h = jnp.maximum((s0_ref[...] + s1_ref[...] + g_ref[...]) * dinv
                    + b_ref[...], 0.0)
    out_ref[...] = jnp.dot(h, w_ref[...],
                           preferred_element_type=jnp.float32) * dinv


def _tc_final_body(s0_ref, s1_ref, g_ref, dinv_ref, b_ref, out_ref):
    out_ref[...] = ((s0_ref[...] + s1_ref[...] + g_ref[...]) * dinv_ref[...]
                    + b_ref[...])


_f32 = jnp.float32
_tc_first = pl.pallas_call(
    _tc_first_body,
    out_shape=[jax.ShapeDtypeStruct((PK, 128), _f32),
               jax.ShapeDtypeStruct((PK, 128), _f32)])
_tc_mid = pl.pallas_call(
    _tc_mid_body, out_shape=jax.ShapeDtypeStruct((PK, 128), _f32))
_tc_final = pl.pallas_call(
    _tc_final_body, out_shape=jax.ShapeDtypeStruct((PK, 128), _f32))


def _blockdiag(w):
    k = w.shape[0]
    return jnp.einsum("pq,kj->pkqj", jnp.eye(8, dtype=w.dtype),
                      w).reshape(8 * k, 128)


def kernel(x, edge_index, W0, b0, W1, b1, W2, b2, W3, b3, W4, b4, W5, b5,
           W6, b6, W7, b7):
    Ws = [W0, W1, W2, W3, W4, W5, W6, W7]
    bs = [b0, b1, b2, b3, b4, b5, b6, b7]

    src = edge_index[0]
    dst = edge_index[1]
    pad = EPAD - E
    srcs = jnp.concatenate(
        [src, jnp.full((pad,), DUMMY, jnp.int32)]).reshape(NTILES, NCH, CH)
    dsts = jnp.concatenate(
        [dst, jnp.full((pad,), DUMMY, jnp.int32)]).reshape(NTILES, NCH, CH)
    zeros = jnp.zeros((NP, D), _f32)
    ones = jnp.ones((NP, D), _f32)
    x_pp = jnp.pad(x, ((0, NP - N), (0, 0))).reshape(PK, 1024)

    w0big = _blockdiag(W0)
    wbigs = [_blockdiag(w) for w in Ws[1:]]
    btiles = [jnp.tile(b, 8).reshape(1, 128) for b in bs]

    aggones = _propagate(ones, srcs, dsts, zeros).reshape(2, PK, 128)

    dinv_p, g_p = _tc_first(aggones[0], aggones[1], x_pp, w0big)

    for i in range(8):
        sp = _propagate(g_p.reshape(NP, D), srcs, dsts, zeros)
        sp = sp.reshape(2, PK, 128)
        if i < 7:
            g_p = _tc_mid(sp[0], sp[1], g_p, dinv_p, btiles[i], wbigs[i])
        else:
            out_p = _tc_final(sp[0], sp[1], g_p, dinv_p, btiles[i])

    return out_p.reshape(NP, D)[:N]


# speedup vs baseline: 28.0566x; 1.4742x over previous
import functools

import jax
import jax.numpy as jnp
from jax import lax
from jax.experimental import pallas as pl
from jax.experimental.pallas import tpu as pltpu
from jax.experimental.pallas import tpu_sc as plsc

N = 10000
E = 320000
D = 16
NP = 10240
NTILES = 32
CH = 128
NCH = 80
EPT = NCH * CH
EPAD = NTILES * EPT
DUMMY = N + 16
RPT = NP // 16
PK = NP // 8


NBUF = 4
NT = NCH // NBUF


def _sc_propagate_body(g_hbm, srcs_hbm, dsts_hbm, zeros_hbm, out_hbm,
                       src_v, dst_v, msg_v, agg_sh, gsem, ssem):
    c = lax.axis_index("c")
    s = lax.axis_index("s")
    tid = s * 2 + c
    pltpu.sync_copy(srcs_hbm.at[tid], src_v)
    pltpu.sync_copy(dsts_hbm.at[tid], dst_v)
    pltpu.sync_copy(zeros_hbm.at[pl.ds(s * RPT, RPT)],
                    agg_sh.at[pl.ds(s * RPT, RPT)])
    for b in range(NBUF):
        pltpu.async_copy(g_hbm.at[src_v.at[b]], msg_v.at[b], gsem)
    plsc.subcore_barrier()

    def body(t, carry):
        for b in range(NBUF):
            j = t * NBUF + b
            pltpu.make_async_copy(g_hbm.at[src_v.at[j]], msg_v.at[b],
                                  gsem).wait()
            pltpu.async_copy(msg_v.at[b], agg_sh.at[dst_v.at[j]], ssem,
                             add=True)

            @pl.when(t < NT - 1)
            def _():
                pltpu.make_async_copy(msg_v.at[b], agg_sh.at[dst_v.at[j]],
                                      ssem).wait()
                pltpu.async_copy(g_hbm.at[src_v.at[j + NBUF]], msg_v.at[b],
                                 gsem)
        return carry

    lax.fori_loop(0, NT, body, 0)
    for b in range(NBUF):
        pltpu.make_async_copy(msg_v.at[b], agg_sh.at[dst_v.at[NCH - NBUF + b]],
                              ssem).wait()
    plsc.subcore_barrier()
    pltpu.sync_copy(agg_sh.at[pl.ds(s * RPT, RPT)],
                    out_hbm.at[c].at[pl.ds(s * RPT, RPT)])


_sc_propagate = functools.partial(
    pl.kernel,
    out_type=jax.ShapeDtypeStruct((2, NP, D), jnp.float32),
    mesh=plsc.VectorSubcoreMesh(core_axis_name="c", subcore_axis_name="s"),
    scratch_types=[
        pltpu.VMEM((NCH, CH), jnp.int32),
        pltpu.VMEM((NCH, CH), jnp.int32),
        pltpu.VMEM((NBUF, CH, D), jnp.float32),
        pltpu.VMEM_SHARED((NP, D), jnp.float32),
        pltpu.SemaphoreType.DMA,
        pltpu.SemaphoreType.DMA,
    ],
    compiler_params=pltpu.CompilerParams(use_tc_tiling_on_sc=False),
)(_sc_propagate_body)


def _propagate(g, srcs, dsts, zeros):
    return _sc_propagate(g, srcs, dsts, zeros)


def _tc_first_body(a0_ref, a1_ref, x_ref, w_ref, dinv_ref, g_ref):
    dinv = lax.rsqrt(a0_ref[...] + a1_ref[...] + 1.0)
    dinv_ref[...] = dinv
    g_ref[...] = jnp.dot(x_ref[...], w_ref[...],
                         preferred_element_type=jnp.float32) * dinv


def _tc_mid_body(s0_ref, s1_ref, g_ref, dinv_ref, b_ref, w_ref, out_ref):
    dinv = dinv_ref[...]
    h = jnp.maximum((s0_ref[...] + s1_ref[...] + g_ref[...]) * dinv
                    + b_ref[...], 0.0)
    out_ref[...] = jnp.dot(h, w_ref[...],
                           preferred_element_type=jnp.float32) * dinv


def _tc_final_body(s0_ref, s1_ref, g_ref, dinv_ref, b_ref, out_ref):
    out_ref[...] = ((s0_ref[...] + s1_ref[...] + g_ref[...]) * dinv_ref[...]
                    + b_ref[...])


_f32 = jnp.float32
_tc_first = pl.pallas_call(
    _tc_first_body,
    out_shape=[jax.ShapeDtypeStruct((PK, 128), _f32),
               jax.ShapeDtypeStruct((PK, 128), _f32)])
_tc_mid = pl.pallas_call(
    _tc_mid_body, out_shape=jax.ShapeDtypeStruct((PK, 128), _f32))
_tc_final = pl.pallas_call(
    _tc_final_body, out_shape=jax.ShapeDtypeStruct((PK, 128), _f32))


def _blockdiag(w):
    k = w.shape[0]
    return jnp.einsum("pq,kj->pkqj", jnp.eye(8, dtype=w.dtype),
                      w).reshape(8 * k, 128)


def kernel(x, edge_index, W0, b0, W1, b1, W2, b2, W3, b3, W4, b4, W5, b5,
           W6, b6, W7, b7):
    Ws = [W0, W1, W2, W3, W4, W5, W6, W7]
    bs = [b0, b1, b2, b3, b4, b5, b6, b7]

    src = edge_index[0]
    dst = edge_index[1]
    pad = EPAD - E
    srcs = jnp.concatenate(
        [src, jnp.full((pad,), DUMMY, jnp.int32)]).reshape(NTILES, NCH, CH)
    dsts = jnp.concatenate(
        [dst, jnp.full((pad,), DUMMY, jnp.int32)]).reshape(NTILES, NCH, CH)
    zeros = jnp.zeros((NP, D), _f32)
    ones = jnp.ones((NP, D), _f32)
    x_pp = jnp.pad(x, ((0, NP - N), (0, 0))).reshape(PK, 1024)

    w0big = _blockdiag(W0)
    wbigs = [_blockdiag(w) for w in Ws[1:]]
    btiles = [jnp.tile(b, 8).reshape(1, 128) for b in bs]

    aggones = _propagate(ones, srcs, dsts, zeros).reshape(2, PK, 128)

    dinv_p, g_p = _tc_first(aggones[0], aggones[1], x_pp, w0big)

    for i in range(8):
        sp = _propagate(g_p.reshape(NP, D), srcs, dsts, zeros)
        sp = sp.reshape(2, PK, 128)
        if i < 7:
            g_p = _tc_mid(sp[0], sp[1], g_p, dinv_p, btiles[i], wbigs[i])
        else:
            out_p = _tc_final(sp[0], sp[1], g_p, dinv_p, btiles[i])

    return out_p.reshape(NP, D)[:N]


# speedup vs baseline: 28.4954x; 1.0156x over previous
import functools

import jax
import jax.numpy as jnp
from jax import lax
from jax.experimental import pallas as pl
from jax.experimental.pallas import tpu as pltpu
from jax.experimental.pallas import tpu_sc as plsc

N = 10000
E = 320000
D = 16
NP = 10240
NTILES = 32
CH = 128
NCH = 80
EPT = NCH * CH
EPAD = NTILES * EPT
DUMMY = N + 16
RPT = NP // 16
PK = NP // 8


NBUF = 4
CG = 4
NG = NCH // CG
NT = NG // NBUF


def _sc_propagate_body(g_hbm, srcs_hbm, dsts_hbm, zeros_hbm, out_hbm,
                       src_v, dst_v, msg_v, agg_sh, gsem, ssem):
    c = lax.axis_index("c")
    s = lax.axis_index("s")
    tid = s * 2 + c
    pltpu.sync_copy(srcs_hbm.at[tid], src_v)
    pltpu.sync_copy(dsts_hbm.at[tid], dst_v)
    pltpu.sync_copy(zeros_hbm.at[pl.ds(s * RPT, RPT)],
                    agg_sh.at[pl.ds(s * RPT, RPT)])
    for b in range(NBUF):
        pltpu.async_copy(g_hbm.at[src_v.at[b]], msg_v.at[b], gsem)
    plsc.subcore_barrier()

    def body(t, carry):
        for b in range(NBUF):
            j = t * NBUF + b
            pltpu.make_async_copy(g_hbm.at[src_v.at[j]], msg_v.at[b],
                                  gsem).wait()
            pltpu.async_copy(msg_v.at[b], agg_sh.at[dst_v.at[j]], ssem,
                             add=True)

            @pl.when(t < NT - 1)
            def _():
                pltpu.make_async_copy(msg_v.at[b], agg_sh.at[dst_v.at[j]],
                                      ssem).wait()
                pltpu.async_copy(g_hbm.at[src_v.at[j + NBUF]], msg_v.at[b],
                                 gsem)
        return carry

    lax.fori_loop(0, NT, body, 0)
    for b in range(NBUF):
        pltpu.make_async_copy(msg_v.at[b], agg_sh.at[dst_v.at[NG - NBUF + b]],
                              ssem).wait()
    plsc.subcore_barrier()
    pltpu.sync_copy(agg_sh.at[pl.ds(s * RPT, RPT)],
                    out_hbm.at[c].at[pl.ds(s * RPT, RPT)])


_sc_propagate = functools.partial(
    pl.kernel,
    out_type=jax.ShapeDtypeStruct((2, NP, D), jnp.float32),
    mesh=plsc.VectorSubcoreMesh(core_axis_name="c", subcore_axis_name="s"),
    scratch_types=[
        pltpu.VMEM((NG, CG * CH), jnp.int32),
        pltpu.VMEM((NG, CG * CH), jnp.int32),
        pltpu.VMEM((NBUF, CG * CH, D), jnp.float32),
        pltpu.VMEM_SHARED((NP, D), jnp.float32),
        pltpu.SemaphoreType.DMA,
        pltpu.SemaphoreType.DMA,
    ],
    compiler_params=pltpu.CompilerParams(use_tc_tiling_on_sc=False),
)(_sc_propagate_body)


def _propagate(g, srcs, dsts, zeros):
    return _sc_propagate(g, srcs, dsts, zeros)


def _tc_first_body(a0_ref, a1_ref, x_ref, w_ref, dinv_ref, g_ref):
    dinv = lax.rsqrt(a0_ref[...] + a1_ref[...] + 1.0)
    dinv_ref[...] = dinv
    g_ref[...] = jnp.dot(x_ref[...], w_ref[...],
                         preferred_element_type=jnp.float32) * dinv


def _tc_mid_body(s0_ref, s1_ref, g_ref, dinv_ref, b_ref, w_ref, out_ref):
    dinv = dinv_ref[...]
    h = jnp.maximum((s0_ref[...] + s1_ref[...] + g_ref[...]) * dinv
                    + b_ref[...], 0.0)
    out_ref[...] = jnp.dot(h, w_ref[...],
                           preferred_element_type=jnp.float32) * dinv


def _tc_final_body(s0_ref, s1_ref, g_ref, dinv_ref, b_ref, out_ref):
    out_ref[...] = ((s0_ref[...] + s1_ref[...] + g_ref[...]) * dinv_ref[...]
                    + b_ref[...])


_f32 = jnp.float32
_tc_first = pl.pallas_call(
    _tc_first_body,
    out_shape=[jax.ShapeDtypeStruct((PK, 128), _f32),
               jax.ShapeDtypeStruct((PK, 128), _f32)])
_tc_mid = pl.pallas_call(
    _tc_mid_body, out_shape=jax.ShapeDtypeStruct((PK, 128), _f32))
_tc_final = pl.pallas_call(
    _tc_final_body, out_shape=jax.ShapeDtypeStruct((PK, 128), _f32))


def _blockdiag(w):
    k = w.shape[0]
    return jnp.einsum("pq,kj->pkqj", jnp.eye(8, dtype=w.dtype),
                      w).reshape(8 * k, 128)


def kernel(x, edge_index, W0, b0, W1, b1, W2, b2, W3, b3, W4, b4, W5, b5,
           W6, b6, W7, b7):
    Ws = [W0, W1, W2, W3, W4, W5, W6, W7]
    bs = [b0, b1, b2, b3, b4, b5, b6, b7]

    src = edge_index[0]
    dst = edge_index[1]
    pad = EPAD - E
    srcs = jnp.concatenate(
        [src, jnp.full((pad,), DUMMY, jnp.int32)]).reshape(NTILES, NG, CG * CH)
    dsts = jnp.concatenate(
        [dst, jnp.full((pad,), DUMMY, jnp.int32)]).reshape(NTILES, NG, CG * CH)
    zeros = jnp.zeros((NP, D), _f32)
    ones = jnp.ones((NP, D), _f32)
    x_pp = jnp.pad(x, ((0, NP - N), (0, 0))).reshape(PK, 1024)

    w0big = _blockdiag(W0)
    wbigs = [_blockdiag(w) for w in Ws[1:]]
    btiles = [jnp.tile(b, 8).reshape(1, 128) for b in bs]

    aggones = _propagate(ones, srcs, dsts, zeros).reshape(2, PK, 128)

    dinv_p, g_p = _tc_first(aggones[0], aggones[1], x_pp, w0big)

    for i in range(8):
        sp = _propagate(g_p.reshape(NP, D), srcs, dsts, zeros)
        sp = sp.reshape(2, PK, 128)
        if i < 7:
            g_p = _tc_mid(sp[0], sp[1], g_p, dinv_p, btiles[i], wbigs[i])
        else:
            out_p = _tc_final(sp[0], sp[1], g_p, dinv_p, btiles[i])

    return out_p.reshape(NP, D)[:N]


# speedup vs baseline: 50.3914x; 1.7684x over previous
import functools

import jax
import jax.numpy as jnp
from jax import lax
from jax.experimental import pallas as pl
from jax.experimental.pallas import tpu as pltpu
from jax.experimental.pallas import tpu_sc as plsc

N = 10000
E = 320000
D = 16
NP = 10240
NTILES = 32
CH = 128
NCH = 80
EPT = NCH * CH
EPAD = NTILES * EPT
DUMMY = N + 16
RPT = NP // 16
PK = NP // 8


NBUF = 4
CG = 4
NG = NCH // CG
NT = NG // NBUF


def _sc_propagate_body(g_hbm, srcs_hbm, dsts_hbm, zeros_hbm, out_hbm,
                       src_v, dst_v, msg_v, g_sh, agg_sh, gsem, ssem):
    c = lax.axis_index("c")
    s = lax.axis_index("s")
    tid = s * 2 + c
    pltpu.sync_copy(srcs_hbm.at[tid], src_v)
    pltpu.sync_copy(dsts_hbm.at[tid], dst_v)
    pltpu.sync_copy(g_hbm.at[pl.ds(s * RPT, RPT)],
                    g_sh.at[pl.ds(s * RPT, RPT)])
    pltpu.sync_copy(zeros_hbm.at[pl.ds(s * RPT, RPT)],
                    agg_sh.at[pl.ds(s * RPT, RPT)])
    plsc.subcore_barrier()
    for b in range(NBUF):
        pltpu.async_copy(g_sh.at[src_v.at[b]], msg_v.at[b], gsem)

    def body(t, carry):
        for b in range(NBUF):
            j = t * NBUF + b
            pltpu.make_async_copy(g_sh.at[src_v.at[j]], msg_v.at[b],
                                  gsem).wait()
            pltpu.async_copy(msg_v.at[b], agg_sh.at[dst_v.at[j]], ssem,
                             add=True)

            @pl.when(t < NT - 1)
            def _():
                pltpu.make_async_copy(msg_v.at[b], agg_sh.at[dst_v.at[j]],
                                      ssem).wait()
                pltpu.async_copy(g_sh.at[src_v.at[j + NBUF]], msg_v.at[b],
                                 gsem)
        return carry

    lax.fori_loop(0, NT, body, 0)
    for b in range(NBUF):
        pltpu.make_async_copy(msg_v.at[b], agg_sh.at[dst_v.at[NG - NBUF + b]],
                              ssem).wait()
    plsc.subcore_barrier()
    pltpu.sync_copy(agg_sh.at[pl.ds(s * RPT, RPT)],
                    out_hbm.at[c].at[pl.ds(s * RPT, RPT)])


_sc_propagate = functools.partial(
    pl.kernel,
    out_type=jax.ShapeDtypeStruct((2, NP, D), jnp.float32),
    mesh=plsc.VectorSubcoreMesh(core_axis_name="c", subcore_axis_name="s"),
    scratch_types=[
        pltpu.VMEM((NG, CG * CH), jnp.int32),
        pltpu.VMEM((NG, CG * CH), jnp.int32),
        pltpu.VMEM((NBUF, CG * CH, D), jnp.float32),
        pltpu.VMEM_SHARED((NP, D), jnp.float32),
        pltpu.VMEM_SHARED((NP, D), jnp.float32),
        pltpu.SemaphoreType.DMA,
        pltpu.SemaphoreType.DMA,
    ],
    compiler_params=pltpu.CompilerParams(use_tc_tiling_on_sc=False),
)(_sc_propagate_body)


def _propagate(g, srcs, dsts, zeros):
    return _sc_propagate(g, srcs, dsts, zeros)


def _tc_first_body(a0_ref, a1_ref, x_ref, w_ref, dinv_ref, g_ref):
    dinv = lax.rsqrt(a0_ref[...] + a1_ref[...] + 1.0)
    dinv_ref[...] = dinv
    g_ref[...] = jnp.dot(x_ref[...], w_ref[...],
                         preferred_element_type=jnp.float32) * dinv


def _tc_mid_body(s0_ref, s1_ref, g_ref, dinv_ref, b_ref, w_ref, out_ref):
    dinv = dinv_ref[...]
    h = jnp.maximum((s0_ref[...] + s1_ref[...] + g_ref[...]) * dinv
                    + b_ref[...], 0.0)
    out_ref[...] = jnp.dot(h, w_ref[...],
                           preferred_element_type=jnp.float32) * dinv


def _tc_final_body(s0_ref, s1_ref, g_ref, dinv_ref, b_ref, out_ref):
    out_ref[...] = ((s0_ref[...] + s1_ref[...] + g_ref[...]) * dinv_ref[...]
                    + b_ref[...])


_f32 = jnp.float32
_tc_first = pl.pallas_call(
    _tc_first_body,
    out_shape=[jax.ShapeDtypeStruct((PK, 128), _f32),
               jax.ShapeDtypeStruct((PK, 128), _f32)])
_tc_mid = pl.pallas_call(
    _tc_mid_body, out_shape=jax.ShapeDtypeStruct((PK, 128), _f32))
_tc_final = pl.pallas_call(
    _tc_final_body, out_shape=jax.ShapeDtypeStruct((PK, 128), _f32))


def _blockdiag(w):
    k = w.shape[0]
    return jnp.einsum("pq,kj->pkqj", jnp.eye(8, dtype=w.dtype),
                      w).reshape(8 * k, 128)


def kernel(x, edge_index, W0, b0, W1, b1, W2, b2, W3, b3, W4, b4, W5, b5,
           W6, b6, W7, b7):
    Ws = [W0, W1, W2, W3, W4, W5, W6, W7]
    bs = [b0, b1, b2, b3, b4, b5, b6, b7]

    src = edge_index[0]
    dst = edge_index[1]
    pad = EPAD - E
    srcs = jnp.concatenate(
        [src, jnp.full((pad,), DUMMY, jnp.int32)]).reshape(NTILES, NG, CG * CH)
    dsts = jnp.concatenate(
        [dst, jnp.full((pad,), DUMMY, jnp.int32)]).reshape(NTILES, NG, CG * CH)
    zeros = jnp.zeros((NP, D), _f32)
    ones = jnp.ones((NP, D), _f32)
    x_pp = jnp.pad(x, ((0, NP - N), (0, 0))).reshape(PK, 1024)

    w0big = _blockdiag(W0)
    wbigs = [_blockdiag(w) for w in Ws[1:]]
    btiles = [jnp.tile(b, 8).reshape(1, 128) for b in bs]

    aggones = _propagate(ones, srcs, dsts, zeros).reshape(2, PK, 128)

    dinv_p, g_p = _tc_first(aggones[0], aggones[1], x_pp, w0big)

    for i in range(8):
        sp = _propagate(g_p.reshape(NP, D), srcs, dsts, zeros)
        sp = sp.reshape(2, PK, 128)
        if i < 7:
            g_p = _tc_mid(sp[0], sp[1], g_p, dinv_p, btiles[i], wbigs[i])
        else:
            out_p = _tc_final(sp[0], sp[1], g_p, dinv_p, btiles[i])

    return out_p.reshape(NP, D)[:N]
